# Initial kernel scaffold; baseline (speedup 1.0000x reference)
import jax
import jax.numpy as jnp
from jax.experimental import pallas as pl


def kernel(x, g1f_wih, g1f_whh, g1f_bih, g1f_bhh, g1b_wih, g1b_whh, g1b_bih, g1b_bhh, g2f_wih, g2f_whh, g2f_bih, g2f_bhh, g2b_wih, g2b_whh, g2b_bih, g2b_bhh, gcn1_W, gcn1_b, gcn2_W, gcn2_b, fc_W, fc_b):
    raise NotImplementedError("write your pallas kernel here")


# speedup vs baseline: 64.3047x; 64.3047x over previous
import functools

import jax
import jax.numpy as jnp
from jax.experimental import pallas as pl

_BLK = 2000
_HALO = 8


def _interleave_cols(wf, wb):
    h = wf.shape[0] // 3
    f = wf.T
    b = wb.T
    return jnp.concatenate(
        [f[:, 0:h], b[:, 0:h], f[:, h:2 * h], b[:, h:2 * h],
         f[:, 2 * h:3 * h], b[:, 2 * h:3 * h]], axis=1)


def _interleave_bias(bf, bb):
    h = bf.shape[0] // 3
    return (jnp.concatenate([bf[0:h], bb[0:h]]),
            jnp.concatenate([bf[h:2 * h], bb[h:2 * h]]),
            jnp.concatenate([bf[2 * h:3 * h], bb[2 * h:3 * h]]))


def _shift_sum(y):
    yp = jnp.pad(y, ((1, 1), (0, 0)))
    return yp[:-2] + yp[1:-1] + yp[2:]


def _fused_body(x_ref, top_ref, bot_ref,
                w1_ref, bi1_ref, br1_ref, bz1_ref, bn1_ref,
                w2_ref, bi2_ref, br2_ref, bz2_ref, bn2_ref,
                gw1_ref, gb1_ref, gw2_ref, gb2_ref, fw_ref, fb_ref,
                out_ref, *, n_real, blk, halo):
    h2 = 128

    def gru(xm, w_ref, bi_ref, br_ref, bz_ref, bn_ref):
        g = jnp.dot(xm, w_ref[...], preferred_element_type=jnp.float32)
        g = g + bi_ref[...]
        r = jax.nn.sigmoid(g[:, 0:h2] + br_ref[...])
        z = jax.nn.sigmoid(g[:, h2:2 * h2] + bz_ref[...])
        n = jnp.tanh(g[:, 2 * h2:3 * h2] + r * bn_ref[...])
        return (1.0 - z) * n

    xe = jnp.concatenate([top_ref[0], x_ref[...], bot_ref[0]], axis=0)
    m = blk + 2 * halo

    seq1 = gru(xe, w1_ref, bi1_ref, br1_ref, bz1_ref, bn1_ref)
    seq2 = gru(seq1, w2_ref, bi2_ref, br2_ref, bz2_ref, bn2_ref)

    i0 = pl.program_id(0) * blk - halo
    idx = i0 + jax.lax.broadcasted_iota(jnp.int32, (m, 1), 0)
    valid = (idx >= 0) & (idx < n_real)
    end = (idx == 0) | (idx == n_real - 1)
    dinv = jnp.where(end, jax.lax.rsqrt(2.0), jax.lax.rsqrt(3.0))
    dv = jnp.where(valid, dinv, 0.0).astype(jnp.float32)

    xw1 = jnp.dot(seq2, gw1_ref[...], preferred_element_type=jnp.float32)
    g1 = dv * _shift_sum(xw1 * dv) + gb1_ref[...]

    xw2 = jnp.dot(g1, gw2_ref[...], preferred_element_type=jnp.float32)
    g2 = dv * _shift_sum(xw2 * dv) + gb2_ref[...]
    g2c = g2[halo:halo + blk]

    out = jnp.dot(g2c, fw_ref[...], preferred_element_type=jnp.float32)
    out_ref[...] = out + fb_ref[...]


@jax.jit
def kernel(x, g1f_wih, g1f_whh, g1f_bih, g1f_bhh, g1b_wih, g1b_whh, g1b_bih, g1b_bhh,
           g2f_wih, g2f_whh, g2f_bih, g2f_bhh, g2b_wih, g2b_whh, g2b_bih, g2b_bhh,
           gcn1_W, gcn1_b, gcn2_W, gcn2_b, fc_W, fc_b):
    n, d = x.shape
    odim = fc_W.shape[1]

    blk = _BLK
    halo = _HALO
    nb = -(-n // blk)
    npad = nb * blk
    if npad != n:
        x = jnp.pad(x, ((0, npad - n), (0, 0)))

    xr = x.reshape(nb, blk, d)
    zrow = jnp.zeros((1, halo, d), dtype=x.dtype)
    tops = jnp.concatenate([zrow, xr[:-1, blk - halo:, :]], axis=0)
    bots = jnp.concatenate([xr[1:, :halo, :], zrow], axis=0)

    w1 = _interleave_cols(g1f_wih, g1b_wih)
    bi1r, bi1z, bi1n = _interleave_bias(g1f_bih, g1b_bih)
    bi1 = jnp.concatenate([bi1r, bi1z, bi1n])[None, :]
    br1, bz1, bn1 = (v[None, :] for v in _interleave_bias(g1f_bhh, g1b_bhh))
    w2 = _interleave_cols(g2f_wih, g2b_wih)
    bi2r, bi2z, bi2n = _interleave_bias(g2f_bih, g2b_bih)
    bi2 = jnp.concatenate([bi2r, bi2z, bi2n])[None, :]
    br2, bz2, bn2 = (v[None, :] for v in _interleave_bias(g2f_bhh, g2b_bhh))

    gb1 = gcn1_b[None, :]
    gb2 = gcn2_b[None, :]
    fb = fc_b[None, :]

    def full(a):
        return pl.BlockSpec(a.shape, lambda i: (0,) * a.ndim)

    body = functools.partial(_fused_body, n_real=n, blk=blk, halo=halo)
    out = pl.pallas_call(
        body,
        grid=(nb,),
        in_specs=[
            pl.BlockSpec((blk, d), lambda i: (i, 0)),
            pl.BlockSpec((1, halo, d), lambda i: (i, 0, 0)),
            pl.BlockSpec((1, halo, d), lambda i: (i, 0, 0)),
            full(w1), full(bi1), full(br1), full(bz1), full(bn1),
            full(w2), full(bi2), full(br2), full(bz2), full(bn2),
            full(gcn1_W), full(gb1), full(gcn2_W), full(gb2),
            full(fc_W), full(fb),
        ],
        out_specs=pl.BlockSpec((blk, odim), lambda i: (i, 0)),
        out_shape=jax.ShapeDtypeStruct((npad, odim), jnp.float32),
    )(x, tops, bots,
      w1, bi1, br1, bz1, bn1,
      w2, bi2, br2, bz2, bn2,
      gcn1_W, gb1, gcn2_W, gb2, fc_W, fb)

    if npad != n:
        out = out[:n]
    return out


# speedup vs baseline: 66.6987x; 1.0372x over previous
import functools

import jax
import jax.numpy as jnp
from jax.experimental import pallas as pl

_BLK = 2000
_HALO = 8


def _interleave_cols(wf, wb):
    h = wf.shape[0] // 3
    f = wf.T
    b = wb.T
    return jnp.concatenate(
        [f[:, 0:h], b[:, 0:h], f[:, h:2 * h], b[:, h:2 * h],
         f[:, 2 * h:3 * h], b[:, 2 * h:3 * h]], axis=1)


def _interleave_bias(bf, bb):
    h = bf.shape[0] // 3
    return (jnp.concatenate([bf[0:h], bb[0:h]]),
            jnp.concatenate([bf[h:2 * h], bb[h:2 * h]]),
            jnp.concatenate([bf[2 * h:3 * h], bb[2 * h:3 * h]]))


def _shift_sum(y):
    yp = jnp.pad(y, ((1, 1), (0, 0)))
    return yp[:-2] + yp[1:-1] + yp[2:]


def _fused_body(x_ref, top_ref, bot_ref,
                w1_ref, bi1_ref, br1_ref, bz1_ref, bn1_ref,
                w2_ref, bi2_ref, br2_ref, bz2_ref, bn2_ref,
                w12_ref, c1_ref, c2_ref,
                out_ref, *, n_real, blk, halo):
    h2 = 128

    def gru(xm, w_ref, bi_ref, br_ref, bz_ref, bn_ref):
        g = jnp.dot(xm, w_ref[...], preferred_element_type=jnp.float32)
        g = g + bi_ref[...]
        r = jax.nn.sigmoid(g[:, 0:h2] + br_ref[...])
        z = jax.nn.sigmoid(g[:, h2:2 * h2] + bz_ref[...])
        n = jnp.tanh(g[:, 2 * h2:3 * h2] + r * bn_ref[...])
        return (1.0 - z) * n

    xe = jnp.concatenate([top_ref[0], x_ref[...], bot_ref[0]], axis=0)
    m = blk + 2 * halo

    seq1 = gru(xe, w1_ref, bi1_ref, br1_ref, bz1_ref, bn1_ref)
    seq2 = gru(seq1, w2_ref, bi2_ref, br2_ref, bz2_ref, bn2_ref)

    i0 = pl.program_id(0) * blk - halo
    idx = i0 + jax.lax.broadcasted_iota(jnp.int32, (m, 1), 0)
    valid = (idx >= 0) & (idx < n_real)
    end = (idx == 0) | (idx == n_real - 1)
    dinv = jnp.where(end, jax.lax.rsqrt(2.0), jax.lax.rsqrt(3.0))
    dv = jnp.where(valid, dinv, 0.0).astype(jnp.float32)

    a = jnp.dot(seq2, w12_ref[...], preferred_element_type=jnp.float32)
    t1 = dv * _shift_sum(a * dv) + c1_ref[...]
    t2 = dv * _shift_sum(t1 * dv) + c2_ref[...]
    out_ref[...] = t2[halo:halo + blk]


@jax.jit
def kernel(x, g1f_wih, g1f_whh, g1f_bih, g1f_bhh, g1b_wih, g1b_whh, g1b_bih, g1b_bhh,
           g2f_wih, g2f_whh, g2f_bih, g2f_bhh, g2b_wih, g2b_whh, g2b_bih, g2b_bhh,
           gcn1_W, gcn1_b, gcn2_W, gcn2_b, fc_W, fc_b):
    n, d = x.shape
    odim = fc_W.shape[1]

    blk = _BLK
    halo = _HALO
    nb = -(-n // blk)
    npad = nb * blk
    if npad != n:
        x = jnp.pad(x, ((0, npad - n), (0, 0)))

    xr = x.reshape(nb, blk, d)
    zrow = jnp.zeros((1, halo, d), dtype=x.dtype)
    tops = jnp.concatenate([zrow, xr[:-1, blk - halo:, :]], axis=0)
    bots = jnp.concatenate([xr[1:, :halo, :], zrow], axis=0)

    w1 = _interleave_cols(g1f_wih, g1b_wih)
    bi1r, bi1z, bi1n = _interleave_bias(g1f_bih, g1b_bih)
    bi1 = jnp.concatenate([bi1r, bi1z, bi1n])[None, :]
    br1, bz1, bn1 = (v[None, :] for v in _interleave_bias(g1f_bhh, g1b_bhh))
    w2 = _interleave_cols(g2f_wih, g2b_wih)
    bi2r, bi2z, bi2n = _interleave_bias(g2f_bih, g2b_bih)
    bi2 = jnp.concatenate([bi2r, bi2z, bi2n])[None, :]
    br2, bz2, bn2 = (v[None, :] for v in _interleave_bias(g2f_bhh, g2b_bhh))

    w12 = gcn1_W @ gcn2_W @ fc_W
    c1 = (gcn1_b @ gcn2_W @ fc_W)[None, :]
    c2 = (gcn2_b @ fc_W + fc_b)[None, :]

    def full(a):
        return pl.BlockSpec(a.shape, lambda i: (0,) * a.ndim)

    body = functools.partial(_fused_body, n_real=n, blk=blk, halo=halo)
    out = pl.pallas_call(
        body,
        grid=(nb,),
        in_specs=[
            pl.BlockSpec((blk, d), lambda i: (i, 0)),
            pl.BlockSpec((1, halo, d), lambda i: (i, 0, 0)),
            pl.BlockSpec((1, halo, d), lambda i: (i, 0, 0)),
            full(w1), full(bi1), full(br1), full(bz1), full(bn1),
            full(w2), full(bi2), full(br2), full(bz2), full(bn2),
            full(w12), full(c1), full(c2),
        ],
        out_specs=pl.BlockSpec((blk, odim), lambda i: (i, 0)),
        out_shape=jax.ShapeDtypeStruct((npad, odim), jnp.float32),
    )(x, tops, bots,
      w1, bi1, br1, bz1, bn1,
      w2, bi2, br2, bz2, bn2,
      w12, c1, c2)

    if npad != n:
        out = out[:n]
    return out


# speedup vs baseline: 68.0771x; 1.0207x over previous
import functools

import jax
import jax.numpy as jnp
from jax.experimental import pallas as pl

_BLK = 2000
_HALO = 8


def _interleave_cols(wf, wb):
    h = wf.shape[0] // 3
    f = wf.T
    b = wb.T
    return jnp.concatenate(
        [f[:, 0:h], b[:, 0:h], f[:, h:2 * h], b[:, h:2 * h],
         f[:, 2 * h:3 * h], b[:, 2 * h:3 * h]], axis=1)


def _interleave_bias(bf, bb):
    h = bf.shape[0] // 3
    return (jnp.concatenate([bf[0:h], bb[0:h]]),
            jnp.concatenate([bf[h:2 * h], bb[h:2 * h]]),
            jnp.concatenate([bf[2 * h:3 * h], bb[2 * h:3 * h]]))


def _shift_sum(y):
    yp = jnp.pad(y, ((1, 1), (0, 0)))
    return yp[:-2] + yp[1:-1] + yp[2:]


def _fused_body(x_ref, top_ref, bot_ref,
                w1_ref, bi1_ref, br1_ref, bz1_ref, bn1_ref,
                w2_ref, bi2_ref, br2_ref, bz2_ref, bn2_ref,
                w12_ref, c1_ref, c2_ref,
                out_ref, *, n_real, blk, halo):
    h2 = 128

    def gru(xm, w_ref, bi_ref, br_ref, bz_ref, bn_ref):
        g = jnp.dot(xm.astype(jnp.bfloat16), w_ref[...],
                    preferred_element_type=jnp.float32)
        g = g + bi_ref[...]
        r = jax.nn.sigmoid(g[:, 0:h2] + br_ref[...])
        z = jax.nn.sigmoid(g[:, h2:2 * h2] + bz_ref[...])
        n = jnp.tanh(g[:, 2 * h2:3 * h2] + r * bn_ref[...])
        return (1.0 - z) * n

    xe = jnp.concatenate([top_ref[0], x_ref[...], bot_ref[0]], axis=0)
    m = blk + 2 * halo

    seq1 = gru(xe, w1_ref, bi1_ref, br1_ref, bz1_ref, bn1_ref)
    seq2 = gru(seq1, w2_ref, bi2_ref, br2_ref, bz2_ref, bn2_ref)

    i0 = pl.program_id(0) * blk - halo
    idx = i0 + jax.lax.broadcasted_iota(jnp.int32, (m, 1), 0)
    valid = (idx >= 0) & (idx < n_real)
    end = (idx == 0) | (idx == n_real - 1)
    dinv = jnp.where(end, jax.lax.rsqrt(2.0), jax.lax.rsqrt(3.0))
    dv = jnp.where(valid, dinv, 0.0).astype(jnp.float32)

    a = jnp.dot(seq2, w12_ref[...], preferred_element_type=jnp.float32)
    t1 = dv * _shift_sum(a * dv) + c1_ref[...]
    t2 = dv * _shift_sum(t1 * dv) + c2_ref[...]
    out_ref[...] = t2[halo:halo + blk]


@jax.jit
def kernel(x, g1f_wih, g1f_whh, g1f_bih, g1f_bhh, g1b_wih, g1b_whh, g1b_bih, g1b_bhh,
           g2f_wih, g2f_whh, g2f_bih, g2f_bhh, g2b_wih, g2b_whh, g2b_bih, g2b_bhh,
           gcn1_W, gcn1_b, gcn2_W, gcn2_b, fc_W, fc_b):
    n, d = x.shape
    odim = fc_W.shape[1]

    blk = _BLK
    halo = _HALO
    nb = -(-n // blk)
    npad = nb * blk
    if npad != n:
        x = jnp.pad(x, ((0, npad - n), (0, 0)))

    xr = x.reshape(nb, blk, d)
    zrow = jnp.zeros((1, halo, d), dtype=x.dtype)
    tops = jnp.concatenate([zrow, xr[:-1, blk - halo:, :]], axis=0)
    bots = jnp.concatenate([xr[1:, :halo, :], zrow], axis=0)

    w1 = _interleave_cols(g1f_wih, g1b_wih).astype(jnp.bfloat16)
    bi1r, bi1z, bi1n = _interleave_bias(g1f_bih, g1b_bih)
    bi1 = jnp.concatenate([bi1r, bi1z, bi1n])[None, :]
    br1, bz1, bn1 = (v[None, :] for v in _interleave_bias(g1f_bhh, g1b_bhh))
    w2 = _interleave_cols(g2f_wih, g2b_wih).astype(jnp.bfloat16)
    bi2r, bi2z, bi2n = _interleave_bias(g2f_bih, g2b_bih)
    bi2 = jnp.concatenate([bi2r, bi2z, bi2n])[None, :]
    br2, bz2, bn2 = (v[None, :] for v in _interleave_bias(g2f_bhh, g2b_bhh))

    w12 = gcn1_W @ gcn2_W @ fc_W
    c1 = (gcn1_b @ gcn2_W @ fc_W)[None, :]
    c2 = (gcn2_b @ fc_W + fc_b)[None, :]

    def full(a):
        return pl.BlockSpec(a.shape, lambda i: (0,) * a.ndim)

    body = functools.partial(_fused_body, n_real=n, blk=blk, halo=halo)
    out = pl.pallas_call(
        body,
        grid=(nb,),
        in_specs=[
            pl.BlockSpec((blk, d), lambda i: (i, 0)),
            pl.BlockSpec((1, halo, d), lambda i: (i, 0, 0)),
            pl.BlockSpec((1, halo, d), lambda i: (i, 0, 0)),
            full(w1), full(bi1), full(br1), full(bz1), full(bn1),
            full(w2), full(bi2), full(br2), full(bz2), full(bn2),
            full(w12), full(c1), full(c2),
        ],
        out_specs=pl.BlockSpec((blk, odim), lambda i: (i, 0)),
        out_shape=jax.ShapeDtypeStruct((npad, odim), jnp.float32),
    )(x, tops, bots,
      w1, bi1, br1, bz1, bn1,
      w2, bi2, br2, bz2, bn2,
      w12, c1, c2)

    if npad != n:
        out = out[:n]
    return out


# speedup vs baseline: 84.3402x; 1.2389x over previous
import functools

import jax
import jax.numpy as jnp
from jax.experimental import pallas as pl

_BLK = 2000
_HALO = 8


def _interleave_cols(wf, wb, sr, sz, sn):
    h = wf.shape[0] // 3
    f = wf.T
    b = wb.T
    return jnp.concatenate(
        [sr * f[:, 0:h], sr * b[:, 0:h],
         sz * f[:, h:2 * h], sz * b[:, h:2 * h],
         sn * f[:, 2 * h:3 * h], sn * b[:, 2 * h:3 * h]], axis=1)


def _interleave_vec(bf, bb):
    h = bf.shape[0] // 3
    return (jnp.concatenate([bf[0:h], bb[0:h]]),
            jnp.concatenate([bf[h:2 * h], bb[h:2 * h]]),
            jnp.concatenate([bf[2 * h:3 * h], bb[2 * h:3 * h]]))


def _gru_weights(wih_f, bih_f, bhh_f, wih_b, bih_b, bhh_b, in_scale):
    w = _interleave_cols(wih_f * in_scale, wih_b * in_scale, 0.5, -0.5, 1.0)
    bir, biz, bin_ = _interleave_vec(bih_f, bih_b)
    bhr, bhz, bhn = _interleave_vec(bhh_f, bhh_b)
    ball = jnp.concatenate([0.5 * (bir + bhr), -0.5 * (biz + bhz),
                            bin_ + 0.5 * bhn])[None, :]
    cn = (0.5 * bhn)[None, :]
    return w.astype(jnp.bfloat16), ball, cn


def _shift_sum(y):
    yp = jnp.pad(y, ((1, 1), (0, 0)))
    return yp[:-2] + yp[1:-1] + yp[2:]


def _fused_body(x_ref, top_ref, bot_ref,
                w1_ref, ball1_ref, cn1_ref,
                w2_ref, ball2_ref, cn2_ref,
                w12_ref, c1_ref, c2_ref,
                out_ref, *, n_real, blk, halo):
    h2 = 128

    def gru(xm, w_ref, ball_ref, cn_ref):
        g = jnp.dot(xm.astype(jnp.bfloat16), w_ref[...],
                    preferred_element_type=jnp.float32)
        g = g + ball_ref[...]
        rt = jnp.tanh(g[:, 0:h2])
        zt = jnp.tanh(g[:, h2:2 * h2])
        n = jnp.tanh(g[:, 2 * h2:3 * h2] + cn_ref[...] * rt)
        return n * (1.0 + zt)

    xe = jnp.concatenate([top_ref[0], x_ref[...], bot_ref[0]], axis=0)
    m = blk + 2 * halo

    seq1 = gru(xe, w1_ref, ball1_ref, cn1_ref)
    seq2 = gru(seq1, w2_ref, ball2_ref, cn2_ref)

    a = jnp.dot(seq2, w12_ref[...], preferred_element_type=jnp.float32)

    i = pl.program_id(0)
    nb = pl.num_programs(0)
    interior = jnp.logical_and(i > 0, i < nb - 1)

    @pl.when(interior)
    def _():
        t1 = _shift_sum(a) + c1_ref[...]
        t2 = (1.0 / 3.0) * _shift_sum(t1) + c2_ref[...]
        out_ref[...] = t2[halo:halo + blk]

    @pl.when(jnp.logical_not(interior))
    def _():
        i0 = i * blk - halo
        idx = i0 + jax.lax.broadcasted_iota(jnp.int32, (m, 1), 0)
        valid = (idx >= 0) & (idx < n_real)
        end = (idx == 0) | (idx == n_real - 1)
        dinv = jnp.where(end, jax.lax.rsqrt(2.0), jax.lax.rsqrt(3.0))
        dv = jnp.where(valid, dinv, 0.0).astype(jnp.float32)
        dv2 = dv * jnp.sqrt(3.0)
        t1 = dv2 * _shift_sum(a * dv2) + c1_ref[...]
        t2 = dv * _shift_sum(t1 * dv) + c2_ref[...]
        out_ref[...] = t2[halo:halo + blk]


@jax.jit
def kernel(x, g1f_wih, g1f_whh, g1f_bih, g1f_bhh, g1b_wih, g1b_whh, g1b_bih, g1b_bhh,
           g2f_wih, g2f_whh, g2f_bih, g2f_bhh, g2b_wih, g2b_whh, g2b_bih, g2b_bhh,
           gcn1_W, gcn1_b, gcn2_W, gcn2_b, fc_W, fc_b):
    n, d = x.shape
    odim = fc_W.shape[1]

    blk = _BLK
    halo = _HALO
    nb = -(-n // blk)
    npad = nb * blk
    if npad != n:
        x = jnp.pad(x, ((0, npad - n), (0, 0)))

    xr = x.reshape(nb, blk, d)
    zrow = jnp.zeros((1, halo, d), dtype=x.dtype)
    tops = jnp.concatenate([zrow, xr[:-1, blk - halo:, :]], axis=0)
    bots = jnp.concatenate([xr[1:, :halo, :], zrow], axis=0)

    w1, ball1, cn1 = _gru_weights(g1f_wih, g1f_bih, g1f_bhh,
                                  g1b_wih, g1b_bih, g1b_bhh, 1.0)
    w2, ball2, cn2 = _gru_weights(g2f_wih, g2f_bih, g2f_bhh,
                                  g2b_wih, g2b_bih, g2b_bhh, 0.5)
    w12 = (0.5 / 3.0) * (gcn1_W @ gcn2_W @ fc_W)
    c1 = (gcn1_b @ gcn2_W @ fc_W)[None, :]
    c2 = (gcn2_b @ fc_W + fc_b)[None, :]

    def full(a):
        return pl.BlockSpec(a.shape, lambda i: (0,) * a.ndim)

    body = functools.partial(_fused_body, n_real=n, blk=blk, halo=halo)
    out = pl.pallas_call(
        body,
        grid=(nb,),
        in_specs=[
            pl.BlockSpec((blk, d), lambda i: (i, 0)),
            pl.BlockSpec((1, halo, d), lambda i: (i, 0, 0)),
            pl.BlockSpec((1, halo, d), lambda i: (i, 0, 0)),
            full(w1), full(ball1), full(cn1),
            full(w2), full(ball2), full(cn2),
            full(w12), full(c1), full(c2),
        ],
        out_specs=pl.BlockSpec((blk, odim), lambda i: (i, 0)),
        out_shape=jax.ShapeDtypeStruct((npad, odim), jnp.float32),
    )(x, tops, bots,
      w1, ball1, cn1,
      w2, ball2, cn2,
      w12, c1, c2)

    if npad != n:
        out = out[:n]
    return out


# speedup vs baseline: 92.7466x; 1.0997x over previous
import functools

import jax
import jax.numpy as jnp
from jax.experimental import pallas as pl
from jax.experimental.pallas import tpu as pltpu

_BLK = 2000
_HALO = 8


def _interleave_cols(wf, wb, sr, sz, sn):
    h = wf.shape[0] // 3
    f = wf.T
    b = wb.T
    return jnp.concatenate(
        [sr * f[:, 0:h], sr * b[:, 0:h],
         sz * f[:, h:2 * h], sz * b[:, h:2 * h],
         sn * f[:, 2 * h:3 * h], sn * b[:, 2 * h:3 * h]], axis=1)


def _interleave_vec(bf, bb):
    h = bf.shape[0] // 3
    return (jnp.concatenate([bf[0:h], bb[0:h]]),
            jnp.concatenate([bf[h:2 * h], bb[h:2 * h]]),
            jnp.concatenate([bf[2 * h:3 * h], bb[2 * h:3 * h]]))


def _gru_weights(wih_f, bih_f, bhh_f, wih_b, bih_b, bhh_b, in_scale):
    w = _interleave_cols(wih_f * in_scale, wih_b * in_scale, 0.5, -0.5, 1.0)
    bir, biz, bin_ = _interleave_vec(bih_f, bih_b)
    bhr, bhz, bhn = _interleave_vec(bhh_f, bhh_b)
    ball = jnp.concatenate([0.5 * (bir + bhr), -0.5 * (biz + bhz),
                            bin_ + 0.5 * bhn])[None, :]
    cn = (0.5 * bhn)[None, :]
    return w.astype(jnp.bfloat16), ball, cn


def _shift_sum(y):
    yp = jnp.pad(y, ((1, 1), (0, 0)))
    return yp[:-2] + yp[1:-1] + yp[2:]


def _fused_body(x_ref, top_ref, bot_ref,
                w1_ref, ball1_ref, cn1_ref,
                w2_ref, ball2_ref, cn2_ref,
                w12_ref, c1_ref, c2_ref, c12_ref,
                out_ref, a_scr_ref, *, n_real, blk, halo):
    h2 = 128

    def gru(xm, w_ref, ball_ref, cn_ref):
        g = jnp.dot(xm.astype(jnp.bfloat16), w_ref[...],
                    preferred_element_type=jnp.float32)
        g = g + ball_ref[...]
        rt = jnp.tanh(g[:, 0:h2])
        zt = jnp.tanh(g[:, h2:2 * h2])
        n = jnp.tanh(g[:, 2 * h2:3 * h2] + cn_ref[...] * rt)
        return n * (1.0 + zt)

    xe = jnp.concatenate([top_ref[0], x_ref[...], bot_ref[0]], axis=0)
    m = blk + 2 * halo

    seq1 = gru(xe, w1_ref, ball1_ref, cn1_ref)
    seq2 = gru(seq1, w2_ref, ball2_ref, cn2_ref)

    a = jnp.dot(seq2, w12_ref[...], preferred_element_type=jnp.float32)

    i = pl.program_id(0)
    nb = pl.num_programs(0)
    interior = jnp.logical_and(i > 0, i < nb - 1)

    @pl.when(interior)
    def _():
        a_scr_ref[...] = a
        a0 = a_scr_ref[pl.ds(halo, blk), :]
        am1 = a_scr_ref[pl.ds(halo - 1, blk), :]
        ap1 = a_scr_ref[pl.ds(halo + 1, blk), :]
        am2 = a_scr_ref[pl.ds(halo - 2, blk), :]
        ap2 = a_scr_ref[pl.ds(halo + 2, blk), :]
        s1 = a0 + (am1 + ap1)
        s2 = a0 + (am2 + ap2)
        out_ref[...] = (s1 + s1) + (s2 + c12_ref[...])

    @pl.when(jnp.logical_not(interior))
    def _():
        i0 = i * blk - halo
        idx = i0 + jax.lax.broadcasted_iota(jnp.int32, (m, 1), 0)
        valid = (idx >= 0) & (idx < n_real)
        end = (idx == 0) | (idx == n_real - 1)
        dinv = jnp.where(end, jax.lax.rsqrt(2.0), jax.lax.rsqrt(3.0))
        dv = jnp.where(valid, dinv, 0.0).astype(jnp.float32)
        dv3 = dv * 3.0
        t1 = dv3 * _shift_sum(a * dv3) + c1_ref[...]
        t2 = dv * _shift_sum(t1 * dv) + c2_ref[...]
        out_ref[...] = t2[halo:halo + blk]


@jax.jit
def kernel(x, g1f_wih, g1f_whh, g1f_bih, g1f_bhh, g1b_wih, g1b_whh, g1b_bih, g1b_bhh,
           g2f_wih, g2f_whh, g2f_bih, g2f_bhh, g2b_wih, g2b_whh, g2b_bih, g2b_bhh,
           gcn1_W, gcn1_b, gcn2_W, gcn2_b, fc_W, fc_b):
    n, d = x.shape
    odim = fc_W.shape[1]

    blk = _BLK
    halo = _HALO
    nb = -(-n // blk)
    npad = nb * blk
    if npad != n:
        x = jnp.pad(x, ((0, npad - n), (0, 0)))

    xr = x.reshape(nb, blk, d)
    zrow = jnp.zeros((1, halo, d), dtype=x.dtype)
    tops = jnp.concatenate([zrow, xr[:-1, blk - halo:, :]], axis=0)
    bots = jnp.concatenate([xr[1:, :halo, :], zrow], axis=0)

    w1, ball1, cn1 = _gru_weights(g1f_wih, g1f_bih, g1f_bhh,
                                  g1b_wih, g1b_bih, g1b_bhh, 1.0)
    w2, ball2, cn2 = _gru_weights(g2f_wih, g2f_bih, g2f_bhh,
                                  g2b_wih, g2b_bih, g2b_bhh, 0.5)
    w12 = (0.5 / 9.0) * (gcn1_W @ gcn2_W @ fc_W)
    c1 = (gcn1_b @ gcn2_W @ fc_W)[None, :]
    c2 = (gcn2_b @ fc_W + fc_b)[None, :]
    c12 = c1 + c2

    def full(a):
        return pl.BlockSpec(a.shape, lambda i: (0,) * a.ndim)

    body = functools.partial(_fused_body, n_real=n, blk=blk, halo=halo)
    out = pl.pallas_call(
        body,
        grid=(nb,),
        in_specs=[
            pl.BlockSpec((blk, d), lambda i: (i, 0)),
            pl.BlockSpec((1, halo, d), lambda i: (i, 0, 0)),
            pl.BlockSpec((1, halo, d), lambda i: (i, 0, 0)),
            full(w1), full(ball1), full(cn1),
            full(w2), full(ball2), full(cn2),
            full(w12), full(c1), full(c2), full(c12),
        ],
        out_specs=pl.BlockSpec((blk, odim), lambda i: (i, 0)),
        out_shape=jax.ShapeDtypeStruct((npad, odim), jnp.float32),
        scratch_shapes=[pltpu.VMEM((blk + 2 * halo, odim), jnp.float32)],
    )(x, tops, bots,
      w1, ball1, cn1,
      w2, ball2, cn2,
      w12, c1, c2, c12)

    if npad != n:
        out = out[:n]
    return out
